# baseline (speedup 1.0000x reference)
import functools

import jax
import jax.numpy as jnp
from jax import lax
from jax.experimental import pallas as pl
from jax.experimental.pallas import tpu as pltpu
from jax.experimental.pallas import tpu_sc as plsc

BATCH = 16384
EMB = 32
HID = 64

_NC = 2
_NS = 16
_NW = _NC * _NS
_BPW = BATCH // _NW
_LAG = 64


def _gather_body(sidx_h, ridx_h, stab_h, rtab_h, sout_h, rout_h,
                 sidx_v, ridx_v, sem):
    wid = lax.axis_index("s") * _NC + lax.axis_index("c")
    base = wid * _BPW
    pltpu.sync_copy(sidx_h.at[pl.ds(base, _BPW)], sidx_v)
    pltpu.sync_copy(ridx_h.at[pl.ds(base, _BPW)], ridx_v)

    def wait_one():
        pltpu.make_async_copy(
            stab_h.at[pl.ds(0, 1)], sout_h.at[pl.ds(0, 1)], sem).wait()

    def body(g, _):
        svec = sidx_v[pl.ds(g * 16, 16)]
        rvec = ridx_v[pl.ds(g * 16, 16)]
        for l in range(16):
            i = base + g * 16 + l
            pltpu.async_copy(stab_h.at[pl.ds(svec[l], 1)],
                             sout_h.at[pl.ds(i, 1)], sem)
            pltpu.async_copy(rtab_h.at[pl.ds(rvec[l], 1)],
                             rout_h.at[pl.ds(i, 1)], sem)

        @pl.when(g >= _LAG // 32)
        def _():
            for _i in range(32):
                wait_one()
        return 0

    lax.fori_loop(0, _BPW // 16, body, 0)
    for _ in range(_LAG):
        wait_one()


_gather_call = functools.partial(
    pl.kernel,
    out_type=[jax.ShapeDtypeStruct((BATCH, EMB), jnp.float32),
              jax.ShapeDtypeStruct((BATCH, EMB), jnp.float32)],
    mesh=plsc.VectorSubcoreMesh(core_axis_name="c", subcore_axis_name="s"),
    scratch_types=[pltpu.VMEM((_BPW,), jnp.int32),
                   pltpu.VMEM((_BPW,), jnp.int32),
                   pltpu.SemaphoreType.DMA],
    compiler_params=pltpu.CompilerParams(needs_layout_passes=False),
)(_gather_body)


_BLK = 2048


def _mlp_body(s_ref, r_ref, w1s_ref, w1r_ref, b1_ref, w2_ref, b2_ref, out_ref):
    h = jnp.tanh(
        jnp.dot(s_ref[...], w1s_ref[...], preferred_element_type=jnp.float32)
        + jnp.dot(r_ref[...], w1r_ref[...], preferred_element_type=jnp.float32)
        + b1_ref[...])
    logit = jnp.sum(h * w2_ref[...], axis=1) + b2_ref[0, 0]
    out_ref[...] = jax.nn.sigmoid(logit)


def _mlp_call(s_emb, r_emb, w1s, w1r, b1, w2, b2):
    grid = BATCH // _BLK
    return pl.pallas_call(
        _mlp_body,
        grid=(grid,),
        in_specs=[
            pl.BlockSpec((_BLK, EMB), lambda i: (i, 0)),
            pl.BlockSpec((_BLK, EMB), lambda i: (i, 0)),
            pl.BlockSpec((EMB, HID), lambda i: (0, 0)),
            pl.BlockSpec((EMB, HID), lambda i: (0, 0)),
            pl.BlockSpec((1, HID), lambda i: (0, 0)),
            pl.BlockSpec((1, HID), lambda i: (0, 0)),
            pl.BlockSpec((1, 1), lambda i: (0, 0)),
        ],
        out_specs=pl.BlockSpec((_BLK,), lambda i: (i,)),
        out_shape=jax.ShapeDtypeStruct((BATCH,), jnp.float32),
    )(s_emb, r_emb, w1s, w1r, b1, w2, b2)


def kernel(sender_idx_batch, receiver_idx_batch, sender_table, receiver_table,
           W1, b1, W2, b2):
    sidx = sender_idx_batch.astype(jnp.int32)
    ridx = receiver_idx_batch.astype(jnp.int32)
    s_emb, r_emb = _gather_call(sidx, ridx, sender_table, receiver_table)
    w1s = W1[:, :EMB].T
    w1r = W1[:, EMB:].T
    b1r = b1.reshape(1, HID)
    w2r = W2.reshape(1, HID)
    b2r = b2.reshape(1, 1)
    return _mlp_call(s_emb, r_emb, w1s, w1r, b1r, w2r, b2r)


# speedup vs baseline: 1.7885x; 1.7885x over previous
import functools

import jax
import jax.numpy as jnp
from jax import lax
from jax.experimental import pallas as pl
from jax.experimental.pallas import tpu as pltpu
from jax.experimental.pallas import tpu_sc as plsc

BATCH = 16384
EMB = 32
HID = 64

_NC = 2
_NS = 16
_NW = _NC * _NS
_BPW = BATCH // _NW
_LAG = 64


def _gather_one_table(idx_h, tab_h, out_h, wid, idx_v, rows_v, sem):
    base = wid * _BPW
    pltpu.sync_copy(idx_h.at[pl.ds(base, _BPW)], idx_v)

    def wait_one():
        pltpu.make_async_copy(
            tab_h.at[pl.ds(0, 1)], rows_v.at[pl.ds(0, 1)], sem).wait()

    def body(g, _):
        vec = idx_v[pl.ds(g * 16, 16)]
        for l in range(16):
            pltpu.async_copy(tab_h.at[pl.ds(vec[l], 1)],
                             rows_v.at[pl.ds(g * 16 + l, 1)], sem)

        @pl.when(g >= _LAG // 16)
        def _():
            for _i in range(16):
                wait_one()
        return 0

    lax.fori_loop(0, _BPW // 16, body, 0)
    for _ in range(_LAG):
        wait_one()
    pltpu.sync_copy(rows_v, out_h.at[pl.ds(base, _BPW)])


def _gather_body(sidx_h, ridx_h, stab_h, rtab_h, sout_h, rout_h,
                 idx_v, rows_v, sem):
    wid = lax.axis_index("s") * _NC + lax.axis_index("c")
    _gather_one_table(sidx_h, stab_h, sout_h, wid, idx_v, rows_v, sem)
    _gather_one_table(ridx_h, rtab_h, rout_h, wid, idx_v, rows_v, sem)


_gather_call = functools.partial(
    pl.kernel,
    out_type=[jax.ShapeDtypeStruct((BATCH, EMB), jnp.float32),
              jax.ShapeDtypeStruct((BATCH, EMB), jnp.float32)],
    mesh=plsc.VectorSubcoreMesh(core_axis_name="c", subcore_axis_name="s"),
    scratch_types=[pltpu.VMEM((_BPW,), jnp.int32),
                   pltpu.VMEM((_BPW, EMB), jnp.float32),
                   pltpu.SemaphoreType.DMA],
    compiler_params=pltpu.CompilerParams(needs_layout_passes=False),
)(_gather_body)


_BLK = 2048


def _mlp_body(s_ref, r_ref, w1s_ref, w1r_ref, b1_ref, w2_ref, b2_ref, out_ref):
    h = jnp.tanh(
        jnp.dot(s_ref[...], w1s_ref[...], preferred_element_type=jnp.float32)
        + jnp.dot(r_ref[...], w1r_ref[...], preferred_element_type=jnp.float32)
        + b1_ref[...])
    logit = jnp.sum(h * w2_ref[...], axis=1) + b2_ref[0, 0]
    out_ref[...] = jax.nn.sigmoid(logit)


def _mlp_call(s_emb, r_emb, w1s, w1r, b1, w2, b2):
    grid = BATCH // _BLK
    return pl.pallas_call(
        _mlp_body,
        grid=(grid,),
        in_specs=[
            pl.BlockSpec((_BLK, EMB), lambda i: (i, 0)),
            pl.BlockSpec((_BLK, EMB), lambda i: (i, 0)),
            pl.BlockSpec((EMB, HID), lambda i: (0, 0)),
            pl.BlockSpec((EMB, HID), lambda i: (0, 0)),
            pl.BlockSpec((1, HID), lambda i: (0, 0)),
            pl.BlockSpec((1, HID), lambda i: (0, 0)),
            pl.BlockSpec((1, 1), lambda i: (0, 0)),
        ],
        out_specs=pl.BlockSpec((_BLK,), lambda i: (i,)),
        out_shape=jax.ShapeDtypeStruct((BATCH,), jnp.float32),
    )(s_emb, r_emb, w1s, w1r, b1, w2, b2)


def kernel(sender_idx_batch, receiver_idx_batch, sender_table, receiver_table,
           W1, b1, W2, b2):
    sidx = sender_idx_batch.astype(jnp.int32)
    ridx = receiver_idx_batch.astype(jnp.int32)
    s_emb, r_emb = _gather_call(sidx, ridx, sender_table, receiver_table)
    w1s = W1[:, :EMB].T
    w1r = W1[:, EMB:].T
    b1r = b1.reshape(1, HID)
    w2r = W2.reshape(1, HID)
    b2r = b2.reshape(1, 1)
    return _mlp_call(s_emb, r_emb, w1s, w1r, b1r, w2r, b2r)
